# baseline (speedup 1.0000x reference)
import jax
import jax.numpy as jnp
import numpy as np
from jax.experimental import pallas as pl

_B = 8
_C = 192
_H = 56
_NB = 4
_NH = 4
_HD = _C // _NH
_G = 4
_IPG = _C // _G
_HS = _H // 2
_LS = _HS * _HS
_L = _H * _H


def _fused_kernel(xb_ref, x4_ref, wall_ref, cw_ref, cb_ref, a_ref, beff_ref,
                  bb_ref, o_ref):
    f32 = jnp.float32

    mt = jax.lax.dot_general(xb_ref[0], wall_ref[...], (((0,), (1,)), ((), ())),
                             preferred_element_type=f32)
    mt3 = mt.reshape(_HS, 2 * _H, _NB * _C)
    mv = jnp.maximum(mt3[:, :_H, :], mt3[:, _H:, :]).reshape(_HS * _H, _NB * _C)
    nv = jnp.minimum(mt3[:, :_H, :], mt3[:, _H:, :]).reshape(_HS * _H, _NB * _C)
    p2 = jnp.maximum(mv, jnp.roll(mv, -1, axis=0))
    n2 = jnp.minimum(nv, jnp.roll(nv, -1, axis=0))
    srow = jax.lax.broadcasted_iota(jnp.int32, (2 * _LS, 1), 0)
    even = (srow % 2) == 0

    fs = []
    for i in range(_NB):
        acc = None
        for g in range(_G):
            cwg = cw_ref[i, g]
            wwt = jax.lax.dot_general(cwg, cwg, (((1,), (1,)), ((), ())),
                                      preferred_element_type=f32)
            rid = jax.lax.broadcasted_iota(jnp.int32, (_IPG, _IPG), 0)
            cid = jax.lax.broadcasted_iota(jnp.int32, (_IPG, _IPG), 1)
            dif = wwt - jnp.where(rid == cid, f32(1.0), f32(0.0))
            s = jnp.sum(dif * dif)
            acc = s if acc is None else acc + s
        fs.append(f32(1.0) - acc / f32(_G * _IPG * _IPG))

    mrow = jax.lax.broadcasted_iota(jnp.int32, (64, _NH), 0)
    hcol = jax.lax.broadcasted_iota(jnp.int32, (64, _NH), 1)
    smat_h = jnp.where(mrow // 16 == hcol, f32(1.0), f32(0.0))

    wall_f = wall_ref[...].astype(f32)
    K = jnp.zeros((_C, 1), f32)
    F = f32(1.0)
    for i in range(_NB):
        w_i = jnp.concatenate(
            [wall_f[64 * i:64 * i + 64],
             wall_f[256 + 64 * i:256 + 64 * i + 64],
             wall_f[512 + 64 * i:512 + 64 * i + 64]], axis=0)
        mixk = jax.lax.dot_general(K, w_i, (((0,), (1,)), ((), ())),
                                   preferred_element_type=f32) + cb_ref[i]
        pos = F >= 0
        x_u = F * jnp.where(pos, p2[:, 64 * i:64 * i + 64],
                            n2[:, 64 * i:64 * i + 64]) + mixk[:, 0:64]
        x_w = F * jnp.where(pos, p2[:, 256 + 64 * i:256 + 64 * i + 64],
                            n2[:, 256 + 64 * i:256 + 64 * i + 64]) \
            + mixk[:, 64:128]
        x_v = F * jnp.where(pos, p2[:, 512 + 64 * i:512 + 64 * i + 64],
                            n2[:, 512 + 64 * i:512 + 64 * i + 64]) \
            + mixk[:, 128:192]
        prod = x_u * x_w * a_ref[i]
        scores = jnp.dot(prod, smat_h, preferred_element_type=f32)
        scores = jnp.where(even, scores, f32(-1e30))
        mx = jnp.max(scores, axis=0, keepdims=True)
        e = jnp.exp(scores - mx)
        attn = e / jnp.sum(e, axis=0, keepdims=True)
        ws = jax.lax.dot_general(x_v, attn, (((0,), (0,)), ((), ())),
                                 preferred_element_type=f32)
        wsum = jnp.sum(ws * smat_h, axis=1, keepdims=True)
        constv = jnp.dot(beff_ref[i], wsum, preferred_element_type=f32) \
            + bb_ref[i]
        K = constv + fs[i] * K
        F = F * fs[i]

    o_ref[0] = F * x4_ref[0] + K.reshape(_C, 1, 1)


def kernel(x, cluster_w, cluster_b, qkv_w, bais_w, bais_b):
    f32 = jnp.float32
    x = x.astype(f32)
    xb = x.reshape(_B, _C, _L).astype(jnp.bfloat16)

    eye_g = jnp.eye(_G, dtype=f32)
    wbd = jnp.einsum('bgoi,gh->bgohi', cluster_w.astype(f32), eye_g) \
             .reshape(_NB, _C, _C)
    wall = jnp.concatenate(
        [wbd[:, 0:64, :].reshape(_NB * 64, _C),
         wbd[:, 64:128, :].reshape(_NB * 64, _C),
         wbd[:, 128:192, :].reshape(_NB * 64, _C)], axis=0)
    qw = qkv_w.astype(f32)
    aflat = ((qw[:, :_C] * qw[:, _C:2 * _C]).reshape(_NB, 64, 3).sum(-1)
             / np.sqrt(_HD).astype(np.float32))[:, None, :]
    beff = (bais_w.astype(f32) * qw[:, 2 * _C:][:, None, :]) \
        .reshape(_NB, _C, 64, 3).sum(-1)
    cb2 = cluster_b.astype(f32)[:, None, :]
    bb3 = bais_b.astype(f32)[..., None]

    return pl.pallas_call(
        _fused_kernel,
        grid=(_B,),
        in_specs=[
            pl.BlockSpec((1, _C, _L), lambda b: (b, 0, 0)),
            pl.BlockSpec((1, _C, _H, _H), lambda b: (b, 0, 0, 0)),
            pl.BlockSpec((_NB * _C, _C), lambda b: (0, 0)),
            pl.BlockSpec((_NB, _G, _IPG, _IPG), lambda b: (0, 0, 0, 0)),
            pl.BlockSpec((_NB, 1, _C), lambda b: (0, 0, 0)),
            pl.BlockSpec((_NB, 1, 64), lambda b: (0, 0, 0)),
            pl.BlockSpec((_NB, _C, 64), lambda b: (0, 0, 0)),
            pl.BlockSpec((_NB, _C, 1), lambda b: (0, 0, 0)),
        ],
        out_specs=pl.BlockSpec((1, _C, _H, _H), lambda b: (b, 0, 0, 0)),
        out_shape=jax.ShapeDtypeStruct((_B, _C, _H, _H), f32),
    )(xb, x, wall.astype(jnp.bfloat16), cluster_w.astype(f32), cb2, aflat,
      beff, bb3)


# speedup vs baseline: 1.5142x; 1.5142x over previous
import jax
import jax.numpy as jnp
import numpy as np
from jax.experimental import pallas as pl

_B = 8
_C = 192
_H = 56
_NB = 4
_NH = 4
_HD = _C // _NH
_G = 4
_IPG = _C // _G
_HS = _H // 2
_LS = _HS * _HS
_L = _H * _H


def _fused_kernel(x_ref, wall_ref, cw_ref, cb_ref, a_ref, beff_ref,
                  bb_ref, o_ref):
    f32 = jnp.float32
    xv = x_ref[0]

    mt = jax.lax.dot_general(xv.astype(jnp.bfloat16), wall_ref[...],
                             (((0,), (1,)), ((), ())),
                             preferred_element_type=f32)
    mt3 = mt.reshape(_HS, 2 * _H, _NB * _C)
    mv = jnp.maximum(mt3[:, :_H, :], mt3[:, _H:, :]).reshape(_HS * _H, _NB * _C)
    nv = jnp.minimum(mt3[:, :_H, :], mt3[:, _H:, :]).reshape(_HS * _H, _NB * _C)
    p2 = jnp.maximum(mv, jnp.roll(mv, -1, axis=0))
    n2 = jnp.minimum(nv, jnp.roll(nv, -1, axis=0))

    fs = []
    for i in range(_NB):
        acc = None
        for g in range(_G):
            cwg = cw_ref[i, g]
            wwt = jax.lax.dot_general(cwg, cwg, (((1,), (1,)), ((), ())),
                                      preferred_element_type=f32)
            rid = jax.lax.broadcasted_iota(jnp.int32, (_IPG, _IPG), 0)
            cid = jax.lax.broadcasted_iota(jnp.int32, (_IPG, _IPG), 1)
            dif = wwt - jnp.where(rid == cid, f32(1.0), f32(0.0))
            s = jnp.sum(dif * dif)
            acc = s if acc is None else acc + s
        fs.append(f32(1.0) - acc / f32(_G * _IPG * _IPG))

    mrow = jax.lax.broadcasted_iota(jnp.int32, (64, _NH), 0)
    hcol = jax.lax.broadcasted_iota(jnp.int32, (64, _NH), 1)
    smat_h = jnp.where(mrow // 16 == hcol, f32(1.0), f32(0.0))
    srow = jax.lax.broadcasted_iota(jnp.int32, (2 * _LS, 1), 0)
    even = (srow % 2) == 0

    wall_f = wall_ref[...].astype(f32)
    K = jnp.zeros((_C, 1), f32)
    F = f32(1.0)
    for i in range(_NB):
        w_i = jnp.concatenate(
            [wall_f[64 * i:64 * i + 64],
             wall_f[256 + 64 * i:256 + 64 * i + 64],
             wall_f[512 + 64 * i:512 + 64 * i + 64]], axis=0)
        mixk = jax.lax.dot_general(K, w_i, (((0,), (1,)), ((), ())),
                                   preferred_element_type=f32) + cb_ref[i]
        pos = F >= 0
        x_u = F * jnp.where(pos, p2[:, 64 * i:64 * i + 64],
                            n2[:, 64 * i:64 * i + 64]) + mixk[:, 0:64]
        x_w = F * jnp.where(pos, p2[:, 256 + 64 * i:256 + 64 * i + 64],
                            n2[:, 256 + 64 * i:256 + 64 * i + 64]) \
            + mixk[:, 64:128]
        x_v = F * jnp.where(pos, p2[:, 512 + 64 * i:512 + 64 * i + 64],
                            n2[:, 512 + 64 * i:512 + 64 * i + 64]) \
            + mixk[:, 128:192]
        prod = x_u * x_w * a_ref[i]
        scores = jnp.dot(prod, smat_h, preferred_element_type=f32)
        scores = jnp.where(even, scores, f32(-1e30))
        mx = jnp.max(scores, axis=0, keepdims=True)
        e = jnp.exp(scores - mx)
        attn = e / jnp.sum(e, axis=0, keepdims=True)
        ws = jax.lax.dot_general(x_v, attn, (((0,), (0,)), ((), ())),
                                 preferred_element_type=f32)
        wsum = jnp.sum(ws * smat_h, axis=1, keepdims=True)
        constv = jnp.dot(beff_ref[i], wsum, preferred_element_type=f32) \
            + bb_ref[i]
        K = constv + fs[i] * K
        F = F * fs[i]

    o_ref[0] = F * xv + K


def kernel(x, cluster_w, cluster_b, qkv_w, bais_w, bais_b):
    f32 = jnp.float32
    x = x.astype(f32)
    xf = x.reshape(_B, _C, _L)

    eye_g = jnp.eye(_G, dtype=f32)
    wbd = jnp.einsum('bgoi,gh->bgohi', cluster_w.astype(f32), eye_g) \
             .reshape(_NB, _C, _C)
    wall = jnp.concatenate(
        [wbd[:, 0:64, :].reshape(_NB * 64, _C),
         wbd[:, 64:128, :].reshape(_NB * 64, _C),
         wbd[:, 128:192, :].reshape(_NB * 64, _C)], axis=0)
    qw = qkv_w.astype(f32)
    aflat = ((qw[:, :_C] * qw[:, _C:2 * _C]).reshape(_NB, 64, 3).sum(-1)
             / np.sqrt(_HD).astype(np.float32))[:, None, :]
    beff = (bais_w.astype(f32) * qw[:, 2 * _C:][:, None, :]) \
        .reshape(_NB, _C, 64, 3).sum(-1)
    cb2 = cluster_b.astype(f32)[:, None, :]
    bb3 = bais_b.astype(f32)[..., None]

    out = pl.pallas_call(
        _fused_kernel,
        grid=(_B,),
        in_specs=[
            pl.BlockSpec((1, _C, _L), lambda b: (b, 0, 0)),
            pl.BlockSpec((_NB * _C, _C), lambda b: (0, 0)),
            pl.BlockSpec((_NB, _G, _IPG, _IPG), lambda b: (0, 0, 0, 0)),
            pl.BlockSpec((_NB, 1, _C), lambda b: (0, 0, 0)),
            pl.BlockSpec((_NB, 1, 64), lambda b: (0, 0, 0)),
            pl.BlockSpec((_NB, _C, 64), lambda b: (0, 0, 0)),
            pl.BlockSpec((_NB, _C, 1), lambda b: (0, 0, 0)),
        ],
        out_specs=pl.BlockSpec((1, _C, _L), lambda b: (b, 0, 0)),
        out_shape=jax.ShapeDtypeStruct((_B, _C, _L), f32),
    )(xf, wall.astype(jnp.bfloat16), cluster_w.astype(f32), cb2, aflat,
      beff, bb3)
    return out.reshape(_B, _C, _H, _H)


# speedup vs baseline: 1.6983x; 1.1216x over previous
import jax
import jax.numpy as jnp
import numpy as np
from jax.experimental import pallas as pl

_B = 8
_C = 192
_H = 56
_NB = 4
_NH = 4
_HD = _C // _NH
_G = 4
_IPG = _C // _G
_HS = _H // 2
_LS = _HS * _HS
_L = _H * _H


def _fused_kernel(x_ref, wall_ref, wallt_ref, cw_ref, cb_ref, a_ref,
                  befft_ref, bb_ref, o_ref):
    f32 = jnp.float32
    xv = x_ref[0]

    mt = jax.lax.dot_general(xv.astype(jnp.bfloat16), wall_ref[...],
                             (((0,), (1,)), ((), ())),
                             preferred_element_type=f32)
    mt3 = mt.reshape(_HS, 2 * _H, _NB * _C)
    mv = jnp.maximum(mt3[:, :_H, :], mt3[:, _H:, :]).reshape(_HS * _H, _NB * _C)
    nv = jnp.minimum(mt3[:, :_H, :], mt3[:, _H:, :]).reshape(_HS * _H, _NB * _C)
    p2 = jnp.maximum(mv, jnp.roll(mv, -1, axis=0))
    n2 = jnp.minimum(nv, jnp.roll(nv, -1, axis=0))

    fs = []
    for i in range(_NB):
        acc = None
        for g in range(_G):
            cwg = cw_ref[i, g]
            wwt = jax.lax.dot_general(cwg, cwg, (((1,), (1,)), ((), ())),
                                      preferred_element_type=f32)
            rid = jax.lax.broadcasted_iota(jnp.int32, (_IPG, _IPG), 0)
            cid = jax.lax.broadcasted_iota(jnp.int32, (_IPG, _IPG), 1)
            dif = wwt - jnp.where(rid == cid, f32(1.0), f32(0.0))
            s = jnp.sum(dif * dif)
            acc = s if acc is None else acc + s
        fs.append(f32(1.0) - acc / f32(_G * _IPG * _IPG))

    hrow = jax.lax.broadcasted_iota(jnp.int32, (_NH, 64), 0)
    mcol = jax.lax.broadcasted_iota(jnp.int32, (_NH, 64), 1)
    smat_t = jnp.where(mcol // 16 == hrow, f32(1.0), f32(0.0))
    scol = jax.lax.broadcasted_iota(jnp.int32, (1, 2 * _LS), 1)
    even = (scol % 2) == 0
    ir = jax.lax.broadcasted_iota(jnp.int32, (_C, _C), 0)
    ic = jax.lax.broadcasted_iota(jnp.int32, (_C, _C), 1)
    eye_c = jnp.where(ir == ic, f32(1.0), f32(0.0))

    wallt = wallt_ref[...]
    K = jnp.zeros((1, _C), f32)
    F = f32(1.0)
    for i in range(_NB):
        mixk = jnp.dot(K, wallt, preferred_element_type=f32) + cb_ref[i]
        pos = F >= 0
        x_u = F * jnp.where(pos, p2[:, 64 * i:64 * i + 64],
                            n2[:, 64 * i:64 * i + 64]) \
            + mixk[:, 64 * i:64 * i + 64]
        x_w = F * jnp.where(pos, p2[:, 256 + 64 * i:256 + 64 * i + 64],
                            n2[:, 256 + 64 * i:256 + 64 * i + 64]) \
            + mixk[:, 256 + 64 * i:256 + 64 * i + 64]
        x_v = F * jnp.where(pos, p2[:, 512 + 64 * i:512 + 64 * i + 64],
                            n2[:, 512 + 64 * i:512 + 64 * i + 64]) \
            + mixk[:, 512 + 64 * i:512 + 64 * i + 64]
        prod = x_u * x_w * a_ref[i]
        scores = jax.lax.dot_general(smat_t, prod, (((1,), (1,)), ((), ())),
                                     preferred_element_type=f32)
        scores = jnp.where(even, scores, f32(-1e30))
        mx = jnp.max(scores, axis=1, keepdims=True)
        e = jnp.exp(scores - mx)
        attn = e / jnp.sum(e, axis=1, keepdims=True)
        ws = jax.lax.dot_general(attn, x_v, (((1,), (0,)), ((), ())),
                                 preferred_element_type=f32)
        wsum = jnp.sum(ws * smat_t, axis=0, keepdims=True)
        constv = jnp.dot(wsum, befft_ref[i], preferred_element_type=f32) \
            + bb_ref[i]
        K = constv + fs[i] * K
        F = F * fs[i]

    k_col = jax.lax.dot_general(eye_c, K, (((1,), (1,)), ((), ())),
                                preferred_element_type=f32)
    o_ref[0] = F * xv + k_col


def kernel(x, cluster_w, cluster_b, qkv_w, bais_w, bais_b):
    f32 = jnp.float32
    x = x.astype(f32)
    xf = x.reshape(_B, _C, _L)

    eye_g = jnp.eye(_G, dtype=f32)
    wbd = jnp.einsum('bgoi,gh->bgohi', cluster_w.astype(f32), eye_g) \
             .reshape(_NB, _C, _C)
    wall = jnp.concatenate(
        [wbd[:, 0:64, :].reshape(_NB * 64, _C),
         wbd[:, 64:128, :].reshape(_NB * 64, _C),
         wbd[:, 128:192, :].reshape(_NB * 64, _C)], axis=0)
    qw = qkv_w.astype(f32)
    aflat = ((qw[:, :_C] * qw[:, _C:2 * _C]).reshape(_NB, 64, 3).sum(-1)
             / np.sqrt(_HD).astype(np.float32))[:, None, :]
    befft = (bais_w.astype(f32) * qw[:, 2 * _C:][:, None, :]) \
        .reshape(_NB, _C, 64, 3).sum(-1).transpose(0, 2, 1)
    cb_r = jnp.concatenate([jnp.tile(cluster_b.astype(f32)[:, 0:64], (1, _NB)),
                            jnp.tile(cluster_b.astype(f32)[:, 64:128],
                                     (1, _NB)),
                            jnp.tile(cluster_b.astype(f32)[:, 128:192],
                                     (1, _NB))], axis=1)[:, None, :]
    bb2 = bais_b.astype(f32)[:, None, :]

    out = pl.pallas_call(
        _fused_kernel,
        grid=(_B,),
        in_specs=[
            pl.BlockSpec((1, _C, _L), lambda b: (b, 0, 0)),
            pl.BlockSpec((_NB * _C, _C), lambda b: (0, 0)),
            pl.BlockSpec((_C, _NB * _C), lambda b: (0, 0)),
            pl.BlockSpec((_NB, _G, _IPG, _IPG), lambda b: (0, 0, 0, 0)),
            pl.BlockSpec((_NB, 1, _NB * _C), lambda b: (0, 0, 0)),
            pl.BlockSpec((_NB, 1, 64), lambda b: (0, 0, 0)),
            pl.BlockSpec((_NB, 64, _C), lambda b: (0, 0, 0)),
            pl.BlockSpec((_NB, 1, _C), lambda b: (0, 0, 0)),
        ],
        out_specs=pl.BlockSpec((1, _C, _L), lambda b: (b, 0, 0)),
        out_shape=jax.ShapeDtypeStruct((_B, _C, _L), f32),
    )(xf, wall.astype(jnp.bfloat16), wall.T, cluster_w.astype(f32), cb_r,
      aflat, befft, bb2)
    return out.reshape(_B, _C, _H, _H)


# speedup vs baseline: 1.7130x; 1.0087x over previous
import jax
import jax.numpy as jnp
import numpy as np
from jax.experimental import pallas as pl

_B = 8
_C = 192
_H = 56
_NB = 4
_NH = 4
_HD = _C // _NH
_G = 4
_IPG = _C // _G
_HS = _H // 2
_LS = _HS * _HS
_L = _H * _H


def _fused_kernel(x_ref, wall_ref, wallt_ref, cw_ref, cb_ref, a_ref,
                  befft_ref, bb_ref, o_ref):
    f32 = jnp.float32
    xv = x_ref[0]

    mt = jax.lax.dot_general(xv.astype(jnp.bfloat16), wall_ref[...],
                             (((0,), (1,)), ((), ())),
                             preferred_element_type=f32)
    mt3 = mt.reshape(_HS, 2 * _H, _NB * _C)
    mv = jnp.maximum(mt3[:, :_H, :], mt3[:, _H:, :]).reshape(_HS * _H, _NB * _C)
    p2 = jnp.maximum(mv, jnp.roll(mv, -1, axis=0))

    fs = []
    for i in range(_NB):
        acc = None
        for g in range(_G):
            cwg = cw_ref[i, g]
            wwt = jax.lax.dot_general(cwg, cwg, (((1,), (1,)), ((), ())),
                                      preferred_element_type=f32)
            rid = jax.lax.broadcasted_iota(jnp.int32, (_IPG, _IPG), 0)
            cid = jax.lax.broadcasted_iota(jnp.int32, (_IPG, _IPG), 1)
            dif = wwt - jnp.where(rid == cid, f32(1.0), f32(0.0))
            s = jnp.sum(dif * dif)
            acc = s if acc is None else acc + s
        fs.append(f32(1.0) - acc / f32(_G * _IPG * _IPG))

    hrow = jax.lax.broadcasted_iota(jnp.int32, (_NH, 64), 0)
    mcol = jax.lax.broadcasted_iota(jnp.int32, (_NH, 64), 1)
    smat_t = jnp.where(mcol // 16 == hrow, f32(1.0), f32(0.0))
    scol = jax.lax.broadcasted_iota(jnp.int32, (1, 2 * _LS), 1)
    even = (scol % 2) == 0
    ir = jax.lax.broadcasted_iota(jnp.int32, (_C, _C), 0)
    ic = jax.lax.broadcasted_iota(jnp.int32, (_C, _C), 1)
    eye_c = jnp.where(ir == ic, f32(1.0), f32(0.0))

    wallt = wallt_ref[...]

    def chain(p2v, n2v):
        K = jnp.zeros((1, _C), f32)
        F = f32(1.0)
        for i in range(_NB):
            mixk = jnp.dot(K, wallt, preferred_element_type=f32) + cb_ref[i]

            def xds(base):
                lo = base + 64 * i
                if n2v is None:
                    sel = p2v[:, lo:lo + 64]
                else:
                    sel = jnp.where(F >= 0, p2v[:, lo:lo + 64],
                                    n2v[:, lo:lo + 64])
                return F * sel + mixk[:, lo:lo + 64]

            x_u, x_w, x_v = xds(0), xds(256), xds(512)
            prod = x_u * x_w * a_ref[i]
            scores = jax.lax.dot_general(smat_t, prod,
                                         (((1,), (1,)), ((), ())),
                                         preferred_element_type=f32)
            scores = jnp.where(even, scores, f32(-1e30))
            mx = jnp.max(scores, axis=1, keepdims=True)
            e = jnp.exp(scores - mx)
            attn = e / jnp.sum(e, axis=1, keepdims=True)
            ws = jax.lax.dot_general(attn, x_v, (((1,), (0,)), ((), ())),
                                     preferred_element_type=f32)
            wsum = jnp.sum(ws * smat_t, axis=0, keepdims=True)
            constv = jnp.dot(wsum, befft_ref[i],
                             preferred_element_type=f32) + bb_ref[i]
            K = constv + fs[i] * K
            F = F * fs[i]
        return K, F * jnp.ones((1, 1), f32)

    def fast_path(_):
        return chain(p2, None)

    def general_path(_):
        nv = jnp.minimum(mt3[:, :_H, :], mt3[:, _H:, :]) \
                .reshape(_HS * _H, _NB * _C)
        n2 = jnp.minimum(nv, jnp.roll(nv, -1, axis=0))
        return chain(p2, n2)

    pf1 = fs[0]
    pf2 = pf1 * fs[1]
    pf3 = pf2 * fs[2]
    allpos = jnp.logical_and(jnp.logical_and(pf1 >= 0, pf2 >= 0), pf3 >= 0)
    K, Fm = jax.lax.cond(allpos, fast_path, general_path, None)

    k_col = jax.lax.dot_general(eye_c, K, (((1,), (1,)), ((), ())),
                                preferred_element_type=f32)
    o_ref[0] = Fm[0, 0] * xv + k_col


def kernel(x, cluster_w, cluster_b, qkv_w, bais_w, bais_b):
    f32 = jnp.float32
    x = x.astype(f32)
    xf = x.reshape(_B, _C, _L)

    eye_g = jnp.eye(_G, dtype=f32)
    wbd = jnp.einsum('bgoi,gh->bgohi', cluster_w.astype(f32), eye_g) \
             .reshape(_NB, _C, _C)
    wall = jnp.concatenate(
        [wbd[:, 0:64, :].reshape(_NB * 64, _C),
         wbd[:, 64:128, :].reshape(_NB * 64, _C),
         wbd[:, 128:192, :].reshape(_NB * 64, _C)], axis=0)
    qw = qkv_w.astype(f32)
    aflat = ((qw[:, :_C] * qw[:, _C:2 * _C]).reshape(_NB, 64, 3).sum(-1)
             / np.sqrt(_HD).astype(np.float32))[:, None, :]
    befft = (bais_w.astype(f32) * qw[:, 2 * _C:][:, None, :]) \
        .reshape(_NB, _C, 64, 3).sum(-1).transpose(0, 2, 1)
    cb_r = jnp.concatenate([jnp.tile(cluster_b.astype(f32)[:, 0:64], (1, _NB)),
                            jnp.tile(cluster_b.astype(f32)[:, 64:128],
                                     (1, _NB)),
                            jnp.tile(cluster_b.astype(f32)[:, 128:192],
                                     (1, _NB))], axis=1)[:, None, :]
    bb2 = bais_b.astype(f32)[:, None, :]

    out = pl.pallas_call(
        _fused_kernel,
        grid=(_B,),
        in_specs=[
            pl.BlockSpec((1, _C, _L), lambda b: (b, 0, 0)),
            pl.BlockSpec((_NB * _C, _C), lambda b: (0, 0)),
            pl.BlockSpec((_C, _NB * _C), lambda b: (0, 0)),
            pl.BlockSpec((_NB, _G, _IPG, _IPG), lambda b: (0, 0, 0, 0)),
            pl.BlockSpec((_NB, 1, _NB * _C), lambda b: (0, 0, 0)),
            pl.BlockSpec((_NB, 1, 64), lambda b: (0, 0, 0)),
            pl.BlockSpec((_NB, 64, _C), lambda b: (0, 0, 0)),
            pl.BlockSpec((_NB, 1, _C), lambda b: (0, 0, 0)),
        ],
        out_specs=pl.BlockSpec((1, _C, _L), lambda b: (b, 0, 0)),
        out_shape=jax.ShapeDtypeStruct((_B, _C, _L), f32),
    )(xf, wall.astype(jnp.bfloat16), wall.T, cluster_w.astype(f32), cb_r,
      aflat, befft, bb2)
    return out.reshape(_B, _C, _H, _H)
